# baseline (speedup 1.0000x reference)
import functools

import jax
import jax.numpy as jnp
import numpy as np
from jax import lax
from jax.experimental import pallas as pl
from jax.experimental.pallas import tpu as pltpu
from jax.experimental.pallas import tpu_sc as plsc

VOCAB = 100000
D = 64
ROWS_PER_BLOCK = 1000

NC = 2
NS = 16
NW = NC * NS
CHUNK = 512


def _fused_table_body(ph_a_ref, ph_b_ref, su_idx_a_ref, su_idx_b_ref,
                      po_idx_a_ref, po_idx_b_ref, su_tab_ref, po_tab_ref,
                      w_ref, b_ref, out_ref):
    wph = w_ref[0:64, :]
    wsu = w_ref[64:128, :]
    wpo = w_ref[128:192, :]
    sp = jnp.dot(su_tab_ref[...], wsu, preferred_element_type=jnp.float32)
    pp = jnp.dot(po_tab_ref[...], wpo, preferred_element_type=jnp.float32)
    lanes = lax.broadcasted_iota(jnp.int32, (ROWS_PER_BLOCK, 16), 1)

    def half(ph_ref, su_idx_ref, po_idx_ref):
        su_oh = (su_idx_ref[0, 0, :][:, None] == lanes).astype(jnp.float32)
        po_oh = (po_idx_ref[0, 0, :][:, None] == lanes).astype(jnp.float32)
        acc = jnp.dot(ph_ref[...], wph, preferred_element_type=jnp.float32)
        acc += jnp.dot(su_oh, sp, preferred_element_type=jnp.float32)
        acc += jnp.dot(po_oh, pp, preferred_element_type=jnp.float32)
        return acc + b_ref[...]

    acc_a = half(ph_a_ref, su_idx_a_ref, po_idx_a_ref)
    acc_b = half(ph_b_ref, su_idx_b_ref, po_idx_b_ref)
    out_ref[...] = jnp.concatenate([acc_a, acc_b], axis=1)


def _build_fused_table(phoneme_table, sutra_table, position_table,
                       sutra_lookup, position_lookup, proj_w, proj_b):
    nblk = (VOCAB // 2) // ROWS_PER_BLOCK
    su_idx = sutra_lookup.astype(jnp.int32).reshape(2 * nblk, 1, ROWS_PER_BLOCK)
    po_idx = position_lookup.astype(jnp.int32).reshape(2 * nblk, 1, ROWS_PER_BLOCK)
    su_tab = jnp.zeros((16, D), jnp.float32).at[:15].set(sutra_table)
    po_tab = jnp.zeros((16, D), jnp.float32).at[:11].set(position_table)
    return pl.pallas_call(
        _fused_table_body,
        grid=(nblk,),
        in_specs=[
            pl.BlockSpec((ROWS_PER_BLOCK, D), lambda i: (i, 0)),
            pl.BlockSpec((ROWS_PER_BLOCK, D), lambda i: (i + nblk, 0)),
            pl.BlockSpec((1, 1, ROWS_PER_BLOCK), lambda i: (i, 0, 0)),
            pl.BlockSpec((1, 1, ROWS_PER_BLOCK), lambda i: (i + nblk, 0, 0)),
            pl.BlockSpec((1, 1, ROWS_PER_BLOCK), lambda i: (i, 0, 0)),
            pl.BlockSpec((1, 1, ROWS_PER_BLOCK), lambda i: (i + nblk, 0, 0)),
            pl.BlockSpec((16, D), lambda i: (0, 0)),
            pl.BlockSpec((16, D), lambda i: (0, 0)),
            pl.BlockSpec((192, D), lambda i: (0, 0)),
            pl.BlockSpec((1, D), lambda i: (0, 0)),
        ],
        out_specs=pl.BlockSpec((ROWS_PER_BLOCK, 2 * D), lambda i: (i, 0)),
        out_shape=jax.ShapeDtypeStruct((VOCAB // 2, 2 * D), jnp.float32),
    )(phoneme_table, phoneme_table, su_idx, su_idx, po_idx, po_idx,
      su_tab, po_tab, proj_w, proj_b.reshape(1, D))


def _sc_gather_transposed(table, idx2, seq, batch):
    lanes = batch // NW
    mesh = plsc.VectorSubcoreMesh(core_axis_name="c", subcore_axis_name="s",
                                  num_cores=NC, num_subcores=NS)

    @functools.partial(
        pl.kernel,
        mesh=mesh,
        compiler_params=pltpu.CompilerParams(use_tc_tiling_on_sc=False,
                                             needs_layout_passes=False),
        out_type=jax.ShapeDtypeStruct((seq, 8, NW, 8, lanes), jnp.float32),
        scratch_types=[
            pltpu.VMEM((seq, lanes), jnp.int32),
            pltpu.VMEM((lanes, D), jnp.float32),
            pltpu.VMEM((lanes, D), jnp.float32),
            pltpu.VMEM((8, 8, lanes), jnp.float32),
            pltpu.VMEM((8, 8, lanes), jnp.float32),
            pltpu.SemaphoreType.DMA,
            pltpu.SemaphoreType.DMA,
            pltpu.SemaphoreType.DMA,
            pltpu.SemaphoreType.DMA,
        ],
    )
    def gather_kernel(table_hbm, idx_hbm, out_hbm,
                      idx_v, rows0, rows1, t0, t1, gs0, gs1, os0, os1):
        wid = lax.axis_index("s") * NC + lax.axis_index("c")
        pltpu.sync_copy(idx_hbm.at[:, pl.ds(wid * lanes, lanes)], idx_v)

        def transpose_to_tile(rows, t_ref):
            @plsc.parallel_loop(0, 8)
            def tbody(k):
                iot = lax.iota(jnp.int32, 16)
                rv = 16 * k + iot
                for d in range(D):
                    d_v = jnp.zeros((16,), jnp.int32) + d
                    vec = plsc.load_gather(rows, [rv, d_v])
                    t_ref[d // 8, d % 8, pl.ds(16 * k, 16)] = vec

        rows_b = (rows0, rows1)
        t_b = (t0, t1)
        gs_b = (gs0, gs1)
        os_b = (os0, os1)

        def start_gather(s, p):
            pltpu.async_copy(table_hbm.at[idx_v.at[s]], rows_b[p], gs_b[p])

        def wait_gather(p):
            pltpu.make_async_copy(table_hbm.at[idx_v.at[0]], rows_b[p],
                                  gs_b[p]).wait()

        def start_out(s, p):
            pltpu.async_copy(t_b[p], out_hbm.at[s, :, wid], os_b[p])

        def wait_out(p):
            pltpu.make_async_copy(t_b[p], out_hbm.at[0, :, wid],
                                  os_b[p]).wait()

        start_gather(0, 0)
        start_gather(1, 1)
        wait_gather(0)
        transpose_to_tile(rows0, t0)
        start_gather(2, 0)
        start_out(0, 0)
        wait_gather(1)
        transpose_to_tile(rows1, t1)
        start_gather(3, 1)
        start_out(1, 1)

        def body(g, carry):
            s = 2 * g
            wait_gather(0)
            wait_out(0)
            transpose_to_tile(rows0, t0)
            start_gather(s + 2, 0)
            start_out(s, 0)
            wait_gather(1)
            wait_out(1)
            transpose_to_tile(rows1, t1)
            start_gather(s + 3, 1)
            start_out(s + 1, 1)
            return carry

        lax.fori_loop(1, seq // 2 - 1, body, 0)

        s = seq - 2
        wait_gather(0)
        wait_out(0)
        transpose_to_tile(rows0, t0)
        start_out(s, 0)
        wait_gather(1)
        wait_out(1)
        transpose_to_tile(rows1, t1)
        start_out(s + 1, 1)
        wait_out(0)
        wait_out(1)

    return gather_kernel(table, idx2)


def kernel(phoneme_indices, phoneme_table, sutra_table, position_table,
           sutra_lookup, position_lookup, proj_w, proj_b):
    batch, seq = phoneme_indices.shape
    packed = _build_fused_table(phoneme_table, sutra_table, position_table,
                                sutra_lookup, position_lookup, proj_w, proj_b)
    fused = packed.reshape(VOCAB, D)
    idx = phoneme_indices.T.astype(jnp.int32)
    idx = 2 * idx - (VOCAB - 1) * (idx >= VOCAB // 2).astype(jnp.int32)
    out5 = _sc_gather_transposed(fused, idx, seq, batch)
    out5 = lax.optimization_barrier(out5)
    return out5.transpose(2, 4, 0, 1, 3).reshape(batch, seq, D)


# speedup vs baseline: 1.0276x; 1.0276x over previous
import functools

import jax
import jax.numpy as jnp
from jax import lax
from jax.experimental import pallas as pl
from jax.experimental.pallas import tpu as pltpu
from jax.experimental.pallas import tpu_sc as plsc

VOCAB = 100000
D = 64
ROWS_PER_BLOCK = 800

NC = 2
NS = 16
NW = NC * NS
CHUNK = 128
NBUF = 8
LOOK = 4


def _fused_table_body(ph_ref, su_idx_ref, po_idx_ref, su_tab_ref, po_tab_ref,
                      w_ref, b_ref, out_ref):
    wph = w_ref[0:64, :]
    wsu = w_ref[64:128, :]
    wpo = w_ref[128:192, :]
    sp = jnp.dot(su_tab_ref[...], wsu, preferred_element_type=jnp.float32)
    pp = jnp.dot(po_tab_ref[...], wpo, preferred_element_type=jnp.float32)
    su_idx = su_idx_ref[0, 0, :]
    po_idx = po_idx_ref[0, 0, :]
    lanes = lax.broadcasted_iota(jnp.int32, (ROWS_PER_BLOCK, 16), 1)
    su_oh = (su_idx[:, None] == lanes).astype(jnp.float32)
    po_oh = (po_idx[:, None] == lanes).astype(jnp.float32)
    acc = jnp.dot(ph_ref[...], wph, preferred_element_type=jnp.float32)
    acc += jnp.dot(su_oh, sp, preferred_element_type=jnp.float32)
    acc += jnp.dot(po_oh, pp, preferred_element_type=jnp.float32)
    out_ref[...] = acc + b_ref[...]


def _build_fused_table(phoneme_table, sutra_table, position_table,
                       sutra_lookup, position_lookup, proj_w, proj_b):
    nblk = VOCAB // ROWS_PER_BLOCK
    su_idx = sutra_lookup.astype(jnp.int32).reshape(nblk, 1, ROWS_PER_BLOCK)
    po_idx = position_lookup.astype(jnp.int32).reshape(nblk, 1, ROWS_PER_BLOCK)
    su_tab = jnp.zeros((16, D), jnp.float32).at[:15].set(sutra_table)
    po_tab = jnp.zeros((16, D), jnp.float32).at[:11].set(position_table)
    return pl.pallas_call(
        _fused_table_body,
        grid=(nblk,),
        in_specs=[
            pl.BlockSpec((ROWS_PER_BLOCK, D), lambda i: (i, 0)),
            pl.BlockSpec((1, 1, ROWS_PER_BLOCK), lambda i: (i, 0, 0)),
            pl.BlockSpec((1, 1, ROWS_PER_BLOCK), lambda i: (i, 0, 0)),
            pl.BlockSpec((16, D), lambda i: (0, 0)),
            pl.BlockSpec((16, D), lambda i: (0, 0)),
            pl.BlockSpec((192, D), lambda i: (0, 0)),
            pl.BlockSpec((1, D), lambda i: (0, 0)),
        ],
        out_specs=pl.BlockSpec((ROWS_PER_BLOCK, D), lambda i: (i, 0)),
        out_shape=jax.ShapeDtypeStruct((VOCAB, D), jnp.float32),
    )(phoneme_table, su_idx, po_idx, su_tab, po_tab, proj_w,
      proj_b.reshape(1, D))


def _sc_gather(table, idx3, batch):
    b_per_w = batch // NW
    nchunk = b_per_w // CHUNK
    nround = nchunk // NBUF
    mesh = plsc.VectorSubcoreMesh(core_axis_name="c", subcore_axis_name="s",
                                  num_cores=NC, num_subcores=NS)

    @functools.partial(
        pl.kernel,
        mesh=mesh,
        compiler_params=pltpu.CompilerParams(use_tc_tiling_on_sc=False),
        out_type=jax.ShapeDtypeStruct((batch, D), jnp.float32),
        scratch_types=(
            [pltpu.VMEM((nchunk, CHUNK), jnp.int32)]
            + [pltpu.VMEM((CHUNK, D), jnp.float32) for _ in range(NBUF)]
            + [pltpu.SemaphoreType.DMA for _ in range(2 * NBUF)]
        ),
    )
    def gather_kernel(table_hbm, idx_hbm, out_hbm, idx_v, *bufs_and_sems):
        rows = bufs_and_sems[:NBUF]
        gsem = bufs_and_sems[NBUF:2 * NBUF]
        osem = bufs_and_sems[2 * NBUF:]
        wid = lax.axis_index("s") * NC + lax.axis_index("c")
        base = wid * b_per_w

        pltpu.sync_copy(idx_hbm.at[wid], idx_v)

        def start_gather(c, p):
            pltpu.async_copy(table_hbm.at[idx_v.at[c]], rows[p], gsem[p])

        def wait_gather(p):
            pltpu.make_async_copy(table_hbm.at[idx_v.at[0]], rows[p],
                                  gsem[p]).wait()

        def start_out(c, p):
            pltpu.async_copy(rows[p],
                             out_hbm.at[pl.ds(base + c * CHUNK, CHUNK)],
                             osem[p])

        def wait_out(p):
            pltpu.make_async_copy(rows[p], out_hbm.at[pl.ds(0, CHUNK)],
                                  osem[p]).wait()

        for p in range(LOOK):
            start_gather(p, p)
        for j in range(NBUF):
            pr = (j + LOOK) % NBUF
            if j < NBUF - LOOK:
                start_gather(j + LOOK, pr)
            else:
                wait_out(pr)
                start_gather(j + LOOK, pr)
            wait_gather(j)
            start_out(j, j)

        def round_body(r, carry):
            c0 = r * NBUF
            for j in range(NBUF):
                pr = (j + LOOK) % NBUF
                wait_out(pr)
                start_gather(c0 + j + LOOK, pr)
                wait_gather(j)
                start_out(c0 + j, j)
            return carry

        lax.fori_loop(1, nround - 1, round_body, 0)

        c0 = (nround - 1) * NBUF
        for j in range(NBUF):
            pr = (j + LOOK) % NBUF
            if j < NBUF - LOOK:
                wait_out(pr)
                start_gather(c0 + j + LOOK, pr)
            wait_gather(j)
            start_out(c0 + j, j)
        for p in range(NBUF):
            wait_out(p)

    return gather_kernel(table, idx3)


def kernel(phoneme_indices, phoneme_table, sutra_table, position_table,
           sutra_lookup, position_lookup, proj_w, proj_b):
    batch, seq = phoneme_indices.shape
    fused = _build_fused_table(phoneme_table, sutra_table, position_table,
                               sutra_lookup, position_lookup, proj_w, proj_b)
    n = batch * seq
    b_per_w = n // NW
    idx3 = phoneme_indices.reshape(NW, b_per_w // CHUNK, CHUNK).astype(jnp.int32)
    out = _sc_gather(fused, idx3, n)
    return out.reshape(batch, seq, D)


# speedup vs baseline: 1.0830x; 1.0539x over previous
import functools

import jax
import jax.numpy as jnp
from jax import lax
from jax.experimental import pallas as pl
from jax.experimental.pallas import tpu as pltpu
from jax.experimental.pallas import tpu_sc as plsc

VOCAB = 100000
D = 64
ROWS_PER_BLOCK = 2000

NC = 2
NS = 16
NW = NC * NS
CHUNK = 128
NBUF = 8
LOOK = 4


def _fused_table_body(ph_ref, su_idx_ref, po_idx_ref, su_tab_ref, po_tab_ref,
                      w_ref, b_ref, out_ref):
    wph = w_ref[0:64, :]
    wsu = w_ref[64:128, :]
    wpo = w_ref[128:192, :]
    sp = jnp.dot(su_tab_ref[...], wsu, preferred_element_type=jnp.float32)
    pp = jnp.dot(po_tab_ref[...], wpo, preferred_element_type=jnp.float32)
    su_idx = su_idx_ref[0, 0, :]
    po_idx = po_idx_ref[0, 0, :]
    lanes = lax.broadcasted_iota(jnp.int32, (ROWS_PER_BLOCK, 16), 1)
    su_oh = (su_idx[:, None] == lanes).astype(jnp.float32)
    po_oh = (po_idx[:, None] == lanes).astype(jnp.float32)
    acc = jnp.dot(ph_ref[...], wph, preferred_element_type=jnp.float32)
    acc += jnp.dot(su_oh, sp, preferred_element_type=jnp.float32)
    acc += jnp.dot(po_oh, pp, preferred_element_type=jnp.float32)
    out_ref[...] = acc + b_ref[...]


def _build_fused_table(phoneme_table, sutra_table, position_table,
                       sutra_lookup, position_lookup, proj_w, proj_b):
    nblk = VOCAB // ROWS_PER_BLOCK
    su_idx = sutra_lookup.astype(jnp.int32).reshape(nblk, 1, ROWS_PER_BLOCK)
    po_idx = position_lookup.astype(jnp.int32).reshape(nblk, 1, ROWS_PER_BLOCK)
    su_tab = jnp.zeros((16, D), jnp.float32).at[:15].set(sutra_table)
    po_tab = jnp.zeros((16, D), jnp.float32).at[:11].set(position_table)
    return pl.pallas_call(
        _fused_table_body,
        grid=(nblk,),
        in_specs=[
            pl.BlockSpec((ROWS_PER_BLOCK, D), lambda i: (i, 0)),
            pl.BlockSpec((1, 1, ROWS_PER_BLOCK), lambda i: (i, 0, 0)),
            pl.BlockSpec((1, 1, ROWS_PER_BLOCK), lambda i: (i, 0, 0)),
            pl.BlockSpec((16, D), lambda i: (0, 0)),
            pl.BlockSpec((16, D), lambda i: (0, 0)),
            pl.BlockSpec((192, D), lambda i: (0, 0)),
            pl.BlockSpec((1, D), lambda i: (0, 0)),
        ],
        out_specs=pl.BlockSpec((ROWS_PER_BLOCK, D), lambda i: (i, 0)),
        out_shape=jax.ShapeDtypeStruct((VOCAB, D), jnp.float32),
    )(phoneme_table, su_idx, po_idx, su_tab, po_tab, proj_w,
      proj_b.reshape(1, D))


def _sc_gather(table, idx3, batch):
    b_per_w = batch // NW
    nchunk = b_per_w // CHUNK
    nround = nchunk // NBUF
    mesh = plsc.VectorSubcoreMesh(core_axis_name="c", subcore_axis_name="s",
                                  num_cores=NC, num_subcores=NS)

    @functools.partial(
        pl.kernel,
        mesh=mesh,
        compiler_params=pltpu.CompilerParams(use_tc_tiling_on_sc=False),
        out_type=jax.ShapeDtypeStruct((batch, D), jnp.float32),
        scratch_types=(
            [pltpu.VMEM((nchunk, CHUNK), jnp.int32)]
            + [pltpu.VMEM((CHUNK, D), jnp.float32) for _ in range(NBUF)]
            + [pltpu.SemaphoreType.DMA for _ in range(2 * NBUF)]
        ),
    )
    def gather_kernel(table_hbm, idx_hbm, out_hbm, idx_v, *bufs_and_sems):
        rows = bufs_and_sems[:NBUF]
        gsem = bufs_and_sems[NBUF:2 * NBUF]
        osem = bufs_and_sems[2 * NBUF:]
        wid = lax.axis_index("s") * NC + lax.axis_index("c")
        base = wid * b_per_w

        pltpu.sync_copy(idx_hbm.at[wid], idx_v)

        def start_gather(c, p):
            pltpu.async_copy(table_hbm.at[idx_v.at[c]], rows[p], gsem[p])

        def wait_gather(p):
            pltpu.make_async_copy(table_hbm.at[idx_v.at[0]], rows[p],
                                  gsem[p]).wait()

        def start_out(c, p):
            pltpu.async_copy(rows[p],
                             out_hbm.at[pl.ds(base + c * CHUNK, CHUNK)],
                             osem[p])

        def wait_out(p):
            pltpu.make_async_copy(rows[p], out_hbm.at[pl.ds(0, CHUNK)],
                                  osem[p]).wait()

        for p in range(LOOK):
            start_gather(p, p)
        for j in range(NBUF):
            pr = (j + LOOK) % NBUF
            if j < NBUF - LOOK:
                start_gather(j + LOOK, pr)
            else:
                wait_out(pr)
                start_gather(j + LOOK, pr)
            wait_gather(j)
            start_out(j, j)

        def round_body(r, carry):
            c0 = r * NBUF
            for j in range(NBUF):
                pr = (j + LOOK) % NBUF
                wait_out(pr)
                start_gather(c0 + j + LOOK, pr)
                wait_gather(j)
                start_out(c0 + j, j)
            return carry

        lax.fori_loop(1, nround - 1, round_body, 0)

        c0 = (nround - 1) * NBUF
        for j in range(NBUF):
            pr = (j + LOOK) % NBUF
            if j < NBUF - LOOK:
                wait_out(pr)
                start_gather(c0 + j + LOOK, pr)
            wait_gather(j)
            start_out(c0 + j, j)
        for p in range(NBUF):
            wait_out(p)

    return gather_kernel(table, idx3)


def kernel(phoneme_indices, phoneme_table, sutra_table, position_table,
           sutra_lookup, position_lookup, proj_w, proj_b):
    batch, seq = phoneme_indices.shape
    fused = _build_fused_table(phoneme_table, sutra_table, position_table,
                               sutra_lookup, position_lookup, proj_w, proj_b)
    n = batch * seq
    b_per_w = n // NW
    idx3 = phoneme_indices.reshape(NW, b_per_w // CHUNK, CHUNK).astype(jnp.int32)
    out = _sc_gather(fused, idx3, n)
    return out.reshape(batch, seq, D)


# speedup vs baseline: 1.0964x; 1.0124x over previous
import functools

import jax
import jax.numpy as jnp
from jax import lax
from jax.experimental import pallas as pl
from jax.experimental.pallas import tpu as pltpu
from jax.experimental.pallas import tpu_sc as plsc

VOCAB = 100000
D = 64
ROWS_PER_BLOCK = 5000

NC = 2
NS = 16
NW = NC * NS
CHUNK = 128
NBUF = 8
LOOK = 4


def _fused_table_body(ph_ref, su_idx_ref, po_idx_ref, su_tab_ref, po_tab_ref,
                      w_ref, b_ref, out_ref):
    wph = w_ref[0:64, :]
    wsu = w_ref[64:128, :]
    wpo = w_ref[128:192, :]
    sp = jnp.dot(su_tab_ref[...], wsu, preferred_element_type=jnp.float32)
    pp = jnp.dot(po_tab_ref[...], wpo, preferred_element_type=jnp.float32)
    su_idx = su_idx_ref[0, 0, :]
    po_idx = po_idx_ref[0, 0, :]
    lanes = lax.broadcasted_iota(jnp.int32, (ROWS_PER_BLOCK, 16), 1)
    su_oh = (su_idx[:, None] == lanes).astype(jnp.float32)
    po_oh = (po_idx[:, None] == lanes).astype(jnp.float32)
    acc = jnp.dot(ph_ref[...], wph, preferred_element_type=jnp.float32)
    acc += jnp.dot(su_oh, sp, preferred_element_type=jnp.float32)
    acc += jnp.dot(po_oh, pp, preferred_element_type=jnp.float32)
    out_ref[...] = acc + b_ref[...]


def _build_fused_table(phoneme_table, sutra_table, position_table,
                       sutra_lookup, position_lookup, proj_w, proj_b):
    nblk = VOCAB // ROWS_PER_BLOCK
    su_idx = sutra_lookup.astype(jnp.int32).reshape(nblk, 1, ROWS_PER_BLOCK)
    po_idx = position_lookup.astype(jnp.int32).reshape(nblk, 1, ROWS_PER_BLOCK)
    su_tab = jnp.zeros((16, D), jnp.float32).at[:15].set(sutra_table)
    po_tab = jnp.zeros((16, D), jnp.float32).at[:11].set(position_table)
    return pl.pallas_call(
        _fused_table_body,
        grid=(nblk,),
        in_specs=[
            pl.BlockSpec((ROWS_PER_BLOCK, D), lambda i: (i, 0)),
            pl.BlockSpec((1, 1, ROWS_PER_BLOCK), lambda i: (i, 0, 0)),
            pl.BlockSpec((1, 1, ROWS_PER_BLOCK), lambda i: (i, 0, 0)),
            pl.BlockSpec((16, D), lambda i: (0, 0)),
            pl.BlockSpec((16, D), lambda i: (0, 0)),
            pl.BlockSpec((192, D), lambda i: (0, 0)),
            pl.BlockSpec((1, D), lambda i: (0, 0)),
        ],
        out_specs=pl.BlockSpec((ROWS_PER_BLOCK, D), lambda i: (i, 0)),
        out_shape=jax.ShapeDtypeStruct((VOCAB, D), jnp.float32),
    )(phoneme_table, su_idx, po_idx, su_tab, po_tab, proj_w,
      proj_b.reshape(1, D))


def _sc_gather(table, idx3, batch):
    b_per_w = batch // NW
    nchunk = b_per_w // CHUNK
    nround = nchunk // NBUF
    mesh = plsc.VectorSubcoreMesh(core_axis_name="c", subcore_axis_name="s",
                                  num_cores=NC, num_subcores=NS)

    @functools.partial(
        pl.kernel,
        mesh=mesh,
        compiler_params=pltpu.CompilerParams(use_tc_tiling_on_sc=False),
        out_type=jax.ShapeDtypeStruct((batch, D), jnp.float32),
        scratch_types=(
            [pltpu.VMEM((nchunk, CHUNK), jnp.int32)]
            + [pltpu.VMEM((CHUNK, D), jnp.float32) for _ in range(NBUF)]
            + [pltpu.SemaphoreType.DMA for _ in range(2 * NBUF)]
        ),
    )
    def gather_kernel(table_hbm, idx_hbm, out_hbm, idx_v, *bufs_and_sems):
        rows = bufs_and_sems[:NBUF]
        gsem = bufs_and_sems[NBUF:2 * NBUF]
        osem = bufs_and_sems[2 * NBUF:]
        wid = lax.axis_index("s") * NC + lax.axis_index("c")
        base = wid * b_per_w

        pltpu.sync_copy(idx_hbm.at[wid], idx_v)

        def start_gather(c, p):
            pltpu.async_copy(table_hbm.at[idx_v.at[c]], rows[p], gsem[p])

        def wait_gather(p):
            pltpu.make_async_copy(table_hbm.at[idx_v.at[0]], rows[p],
                                  gsem[p]).wait()

        def start_out(c, p):
            pltpu.async_copy(rows[p],
                             out_hbm.at[pl.ds(base + c * CHUNK, CHUNK)],
                             osem[p])

        def wait_out(p):
            pltpu.make_async_copy(rows[p], out_hbm.at[pl.ds(0, CHUNK)],
                                  osem[p]).wait()

        for p in range(LOOK):
            start_gather(p, p)
        for j in range(NBUF):
            pr = (j + LOOK) % NBUF
            if j < NBUF - LOOK:
                start_gather(j + LOOK, pr)
            else:
                wait_out(pr)
                start_gather(j + LOOK, pr)
            wait_gather(j)
            start_out(j, j)

        def round_body(r, carry):
            c0 = r * NBUF
            for j in range(NBUF):
                pr = (j + LOOK) % NBUF
                wait_out(pr)
                start_gather(c0 + j + LOOK, pr)
                wait_gather(j)
                start_out(c0 + j, j)
            return carry

        lax.fori_loop(1, nround - 1, round_body, 0)

        c0 = (nround - 1) * NBUF
        for j in range(NBUF):
            pr = (j + LOOK) % NBUF
            if j < NBUF - LOOK:
                wait_out(pr)
                start_gather(c0 + j + LOOK, pr)
            wait_gather(j)
            start_out(c0 + j, j)
        for p in range(NBUF):
            wait_out(p)

    return gather_kernel(table, idx3)


def kernel(phoneme_indices, phoneme_table, sutra_table, position_table,
           sutra_lookup, position_lookup, proj_w, proj_b):
    batch, seq = phoneme_indices.shape
    fused = _build_fused_table(phoneme_table, sutra_table, position_table,
                               sutra_lookup, position_lookup, proj_w, proj_b)
    n = batch * seq
    b_per_w = n // NW
    idx3 = phoneme_indices.reshape(NW, b_per_w // CHUNK, CHUNK).astype(jnp.int32)
    out = _sc_gather(fused, idx3, n)
    return out.reshape(batch, seq, D)


# speedup vs baseline: 1.1148x; 1.0168x over previous
import functools

import jax
import jax.numpy as jnp
from jax import lax
from jax.experimental import pallas as pl
from jax.experimental.pallas import tpu as pltpu
from jax.experimental.pallas import tpu_sc as plsc

VOCAB = 100000
D = 64
ROWS_PER_BLOCK = 10000

NC = 2
NS = 16
NW = NC * NS
CHUNK = 128
NBUF = 8
LOOK = 4


def _fused_table_body(ph_ref, su_idx_ref, po_idx_ref, su_tab_ref, po_tab_ref,
                      w_ref, b_ref, out_ref):
    wph = w_ref[0:64, :]
    wsu = w_ref[64:128, :]
    wpo = w_ref[128:192, :]
    sp = jnp.dot(su_tab_ref[...], wsu, preferred_element_type=jnp.float32)
    pp = jnp.dot(po_tab_ref[...], wpo, preferred_element_type=jnp.float32)
    su_idx = su_idx_ref[0, 0, :]
    po_idx = po_idx_ref[0, 0, :]
    lanes = lax.broadcasted_iota(jnp.int32, (ROWS_PER_BLOCK, 16), 1)
    su_oh = (su_idx[:, None] == lanes).astype(jnp.float32)
    po_oh = (po_idx[:, None] == lanes).astype(jnp.float32)
    acc = jnp.dot(ph_ref[...], wph, preferred_element_type=jnp.float32)
    acc += jnp.dot(su_oh, sp, preferred_element_type=jnp.float32)
    acc += jnp.dot(po_oh, pp, preferred_element_type=jnp.float32)
    out_ref[...] = acc + b_ref[...]


def _build_fused_table(phoneme_table, sutra_table, position_table,
                       sutra_lookup, position_lookup, proj_w, proj_b):
    nblk = VOCAB // ROWS_PER_BLOCK
    su_idx = sutra_lookup.astype(jnp.int32).reshape(nblk, 1, ROWS_PER_BLOCK)
    po_idx = position_lookup.astype(jnp.int32).reshape(nblk, 1, ROWS_PER_BLOCK)
    su_tab = jnp.zeros((16, D), jnp.float32).at[:15].set(sutra_table)
    po_tab = jnp.zeros((16, D), jnp.float32).at[:11].set(position_table)
    return pl.pallas_call(
        _fused_table_body,
        grid=(nblk,),
        in_specs=[
            pl.BlockSpec((ROWS_PER_BLOCK, D), lambda i: (i, 0)),
            pl.BlockSpec((1, 1, ROWS_PER_BLOCK), lambda i: (i, 0, 0)),
            pl.BlockSpec((1, 1, ROWS_PER_BLOCK), lambda i: (i, 0, 0)),
            pl.BlockSpec((16, D), lambda i: (0, 0)),
            pl.BlockSpec((16, D), lambda i: (0, 0)),
            pl.BlockSpec((192, D), lambda i: (0, 0)),
            pl.BlockSpec((1, D), lambda i: (0, 0)),
        ],
        out_specs=pl.BlockSpec((ROWS_PER_BLOCK, D), lambda i: (i, 0)),
        out_shape=jax.ShapeDtypeStruct((VOCAB, D), jnp.float32),
    )(phoneme_table, su_idx, po_idx, su_tab, po_tab, proj_w,
      proj_b.reshape(1, D))


def _sc_gather(table, idx3, batch):
    b_per_w = batch // NW
    nchunk = b_per_w // CHUNK
    nround = nchunk // NBUF
    mesh = plsc.VectorSubcoreMesh(core_axis_name="c", subcore_axis_name="s",
                                  num_cores=NC, num_subcores=NS)

    @functools.partial(
        pl.kernel,
        mesh=mesh,
        compiler_params=pltpu.CompilerParams(use_tc_tiling_on_sc=False),
        out_type=jax.ShapeDtypeStruct((batch, D), jnp.float32),
        scratch_types=(
            [pltpu.VMEM((nchunk, CHUNK), jnp.int32)]
            + [pltpu.VMEM((CHUNK, D), jnp.float32) for _ in range(NBUF)]
            + [pltpu.SemaphoreType.DMA for _ in range(2 * NBUF)]
        ),
    )
    def gather_kernel(table_hbm, idx_hbm, out_hbm, idx_v, *bufs_and_sems):
        rows = bufs_and_sems[:NBUF]
        gsem = bufs_and_sems[NBUF:2 * NBUF]
        osem = bufs_and_sems[2 * NBUF:]
        wid = lax.axis_index("s") * NC + lax.axis_index("c")
        base = wid * b_per_w

        pltpu.sync_copy(idx_hbm.at[wid], idx_v)

        def start_gather(c, p):
            pltpu.async_copy(table_hbm.at[idx_v.at[c]], rows[p], gsem[p])

        def wait_gather(p):
            pltpu.make_async_copy(table_hbm.at[idx_v.at[0]], rows[p],
                                  gsem[p]).wait()

        def start_out(c, p):
            pltpu.async_copy(rows[p],
                             out_hbm.at[pl.ds(base + c * CHUNK, CHUNK)],
                             osem[p])

        def wait_out(p):
            pltpu.make_async_copy(rows[p], out_hbm.at[pl.ds(0, CHUNK)],
                                  osem[p]).wait()

        for p in range(LOOK):
            start_gather(p, p)
        for j in range(NBUF):
            pr = (j + LOOK) % NBUF
            if j < NBUF - LOOK:
                start_gather(j + LOOK, pr)
            else:
                wait_out(pr)
                start_gather(j + LOOK, pr)
            wait_gather(j)
            start_out(j, j)

        def round_body(r, carry):
            c0 = r * NBUF
            for j in range(NBUF):
                pr = (j + LOOK) % NBUF
                wait_out(pr)
                start_gather(c0 + j + LOOK, pr)
                wait_gather(j)
                start_out(c0 + j, j)
            return carry

        lax.fori_loop(1, nround - 1, round_body, 0)

        c0 = (nround - 1) * NBUF
        for j in range(NBUF):
            pr = (j + LOOK) % NBUF
            if j < NBUF - LOOK:
                wait_out(pr)
                start_gather(c0 + j + LOOK, pr)
            wait_gather(j)
            start_out(c0 + j, j)
        for p in range(NBUF):
            wait_out(p)

    return gather_kernel(table, idx3)


def kernel(phoneme_indices, phoneme_table, sutra_table, position_table,
           sutra_lookup, position_lookup, proj_w, proj_b):
    batch, seq = phoneme_indices.shape
    fused = _build_fused_table(phoneme_table, sutra_table, position_table,
                               sutra_lookup, position_lookup, proj_w, proj_b)
    n = batch * seq
    b_per_w = n // NW
    idx3 = phoneme_indices.reshape(NW, b_per_w // CHUNK, CHUNK).astype(jnp.int32)
    out = _sc_gather(fused, idx3, n)
    return out.reshape(batch, seq, D)
